# baseline (speedup 1.0000x reference)
import functools

import jax
import jax.numpy as jnp
from jax import lax
from jax.experimental import pallas as pl
from jax.experimental.pallas import tpu as pltpu
from jax.experimental.pallas import tpu_sc as plsc

N_NODES = 10000
N_EDGES = 320000
D = 128

NC = 2
NS = 16
NW = NC * NS
CH = 128
NCHUNK = 80
HALF = NCHUNK // 2
E_PAD = NW * NCHUNK * CH
N0 = 40
N1 = 2 * NCHUNK - N0
N_PAD = 10240
RPT = N_PAD // NS


def _sc_body(src_hbm, dst_hbm, feat_hbm, partial_hbm, deg_hbm,
             src_v, dst_v, rows_a, rows_b, ones_v, zdeg_v, acc, dacc,
             gs_a, gs_b, ss_a, ss_b):
    cid = lax.axis_index("c")
    sid = lax.axis_index("s")
    wid = sid * NC + cid

    zeros16 = jnp.zeros((16,), jnp.float32)
    zeros16i = jnp.zeros((16,), jnp.int32)
    def _zrow(r, carry):
        for c in range(D // 16):
            rows_a[r, pl.ds(c * 16, 16)] = zeros16
        return carry
    lax.fori_loop(0, CH, _zrow, 0)
    for k in range(CH // 16):
        ones_v[pl.ds(k * 16, 16)] = jnp.ones((16,), jnp.float32)
    def _zdeg(k, carry):
        zdeg_v[pl.ds(k * 16, 16)] = zeros16
        return carry
    lax.fori_loop(0, RPT // 16, _zdeg, 0)
    lanes = jnp.arange(16, dtype=jnp.int32)
    for r in (HALF, HALF + 1):
        for c in range(CH // 16):
            src_v[r, pl.ds(c * 16, 16)] = wid * CH + c * 16 + lanes

    for i in range(RPT // CH):
        pltpu.sync_copy(rows_a, acc.at[pl.ds(sid * RPT + i * CH, CH)])
    pltpu.sync_copy(zdeg_v, dacc.at[pl.ds(sid * RPT, RPT)])
    plsc.subcore_barrier()

    tile_base = jnp.where(cid == 0, sid * N0, NS * N0 + sid * N1)
    n_passes = jnp.where(cid == 0, N0 // HALF, N1 // HALF)

    def _pass(p, pcarry):
        base = tile_base + p * HALF
        pltpu.sync_copy(src_hbm.at[pl.ds(base, HALF)],
                        src_v.at[pl.ds(0, HALF)])
        pltpu.sync_copy(dst_hbm.at[pl.ds(base, HALF)], dst_v)

        pltpu.async_copy(feat_hbm.at[src_v.at[0]], rows_a, gs_a)
        pltpu.async_copy(feat_hbm.at[src_v.at[1]], rows_b, gs_b)

        def _pair(i, carry):
            a = 2 * i
            b = 2 * i + 1
            pltpu.make_async_copy(feat_hbm.at[src_v.at[a]], rows_a,
                                  gs_a).wait()
            pltpu.sync_copy(rows_a, acc.at[dst_v.at[a]], add=True)
            pltpu.sync_copy(ones_v, dacc.at[dst_v.at[a]], add=True)
            pltpu.async_copy(feat_hbm.at[src_v.at[a + 2]], rows_a, gs_a)
            pltpu.make_async_copy(feat_hbm.at[src_v.at[b]], rows_b,
                                  gs_b).wait()
            pltpu.sync_copy(rows_b, acc.at[dst_v.at[b]], add=True)
            pltpu.sync_copy(ones_v, dacc.at[dst_v.at[b]], add=True)
            pltpu.async_copy(feat_hbm.at[src_v.at[b + 2]], rows_b, gs_b)
            return carry
        lax.fori_loop(0, HALF // 2, _pair, 0)
        pltpu.make_async_copy(feat_hbm.at[src_v.at[HALF]], rows_a,
                              gs_a).wait()
        pltpu.make_async_copy(feat_hbm.at[src_v.at[HALF + 1]], rows_b,
                              gs_b).wait()
        return pcarry
    lax.fori_loop(0, n_passes, _pass, 0)
    plsc.subcore_barrier()

    for i in range(RPT // CH):
        pltpu.sync_copy(
            acc.at[pl.ds(sid * RPT + i * CH, CH)],
            partial_hbm.at[pl.ds(cid * N_PAD + sid * RPT + i * CH, CH)])
    pltpu.sync_copy(dacc.at[pl.ds(sid * RPT, RPT)],
                    deg_hbm.at[pl.ds(cid * N_PAD + sid * RPT, RPT)])


_sc_call = pl.kernel(
    _sc_body,
    out_type=(
        jax.ShapeDtypeStruct((NC * N_PAD, D), jnp.float32),
        jax.ShapeDtypeStruct((NC * N_PAD,), jnp.float32),
    ),
    mesh=plsc.VectorSubcoreMesh(core_axis_name="c", subcore_axis_name="s"),
    scratch_types=[
        pltpu.VMEM((HALF + 2, CH), jnp.int32),
        pltpu.VMEM((HALF, CH), jnp.int32),
        pltpu.VMEM((CH, D), jnp.float32),
        pltpu.VMEM((CH, D), jnp.float32),
        pltpu.VMEM((CH,), jnp.float32),
        pltpu.VMEM((RPT,), jnp.float32),
        pltpu.VMEM_SHARED((N_PAD, D), jnp.float32),
        pltpu.VMEM_SHARED((N_PAD,), jnp.float32),
        pltpu.SemaphoreType.DMA,
        pltpu.SemaphoreType.DMA,
        pltpu.SemaphoreType.DMA,
        pltpu.SemaphoreType.DMA,
    ],
)


_BLK = 1000


def _tc_body(p0_ref, p1_ref, d0_ref, d1_ref, x_ref, wn_ref, ws_ref, b_ref,
             o_ref):
    deg = jnp.maximum(d0_ref[...] + d1_ref[...], 1.0)
    h = (p0_ref[...] + p1_ref[...]) / deg
    cdims = (((1,), (1,)), ((), ()))
    h_neigh = lax.dot_general(h, wn_ref[...], cdims,
                              preferred_element_type=jnp.float32)
    h_self = lax.dot_general(x_ref[...], ws_ref[...], cdims,
                             preferred_element_type=jnp.float32)
    o_ref[...] = h_self + h_neigh + b_ref[...]


def kernel(feat, edge_index, W_neigh, W_self, b_self):
    src = edge_index[0].astype(jnp.int32)
    dst = edge_index[1].astype(jnp.int32)
    pad = E_PAD - N_EDGES
    src_p = jnp.concatenate([src, jnp.zeros((pad,), jnp.int32)])
    dst_p = jnp.concatenate([dst, jnp.full((pad,), N_NODES, jnp.int32)])
    src_p = src_p.reshape(NW * NCHUNK, CH)
    dst_p = dst_p.reshape(NW * NCHUNK, CH)

    partial, deg = _sc_call(src_p, dst_p, feat)
    p0 = partial[:N_NODES]
    p1 = partial[N_PAD:N_PAD + N_NODES]
    d0 = deg[:N_NODES].reshape(N_NODES, 1)
    d1 = deg[N_PAD:N_PAD + N_NODES].reshape(N_NODES, 1)

    grid = (N_NODES // _BLK,)
    row_spec = pl.BlockSpec((_BLK, D), lambda i: (i, 0))
    deg_spec = pl.BlockSpec((_BLK, 1), lambda i: (i, 0))
    w_spec = pl.BlockSpec((D, D), lambda i: (0, 0))
    b_spec = pl.BlockSpec((1, D), lambda i: (0, 0))
    out = pl.pallas_call(
        _tc_body,
        grid=grid,
        in_specs=[row_spec, row_spec, deg_spec, deg_spec, row_spec,
                  w_spec, w_spec, b_spec],
        out_specs=row_spec,
        out_shape=jax.ShapeDtypeStruct((N_NODES, D), jnp.float32),
    )(p0, p1, d0, d1, feat, W_neigh, W_self, b_self.reshape(1, D))
    return out


# speedup vs baseline: 1.0481x; 1.0481x over previous
import functools

import jax
import jax.numpy as jnp
from jax import lax
from jax.experimental import pallas as pl
from jax.experimental.pallas import tpu as pltpu
from jax.experimental.pallas import tpu_sc as plsc

N_NODES = 10000
N_EDGES = 320000
D = 128

NC = 2
NS = 16
NW = NC * NS
CH = 128
NCHUNK = 80
HALF = NCHUNK // 2
E_PAD = NW * NCHUNK * CH
N_PAD = 10240
RPT = N_PAD // NS


def _sc_body(src_hbm, dst_hbm, feat_hbm, partial_hbm, deg_hbm,
             src_v, dst_v, rows_a, rows_b, ones_v, zdeg_v, acc, dacc,
             gs_a, gs_b, ss_a, ss_b):
    cid = lax.axis_index("c")
    sid = lax.axis_index("s")
    wid = sid * NC + cid

    zeros16 = jnp.zeros((16,), jnp.float32)
    zeros16i = jnp.zeros((16,), jnp.int32)
    def _zrow(r, carry):
        for c in range(D // 16):
            rows_a[r, pl.ds(c * 16, 16)] = zeros16
        return carry
    lax.fori_loop(0, CH, _zrow, 0)
    for k in range(CH // 16):
        ones_v[pl.ds(k * 16, 16)] = jnp.ones((16,), jnp.float32)
    def _zdeg(k, carry):
        zdeg_v[pl.ds(k * 16, 16)] = zeros16
        return carry
    lax.fori_loop(0, RPT // 16, _zdeg, 0)
    lanes = jnp.arange(16, dtype=jnp.int32)
    for r in (HALF, HALF + 1):
        for c in range(CH // 16):
            src_v[r, pl.ds(c * 16, 16)] = wid * CH + c * 16 + lanes

    for i in range(RPT // CH):
        pltpu.sync_copy(rows_a, acc.at[pl.ds(sid * RPT + i * CH, CH)])
    pltpu.sync_copy(zdeg_v, dacc.at[pl.ds(sid * RPT, RPT)])
    plsc.subcore_barrier()

    for p in range(NCHUNK // HALF):
        base = wid * NCHUNK + p * HALF
        pltpu.sync_copy(src_hbm.at[pl.ds(base, HALF)],
                        src_v.at[pl.ds(0, HALF)])
        pltpu.sync_copy(dst_hbm.at[pl.ds(base, HALF)], dst_v)

        pltpu.async_copy(feat_hbm.at[src_v.at[0]], rows_a, gs_a)
        pltpu.async_copy(feat_hbm.at[src_v.at[1]], rows_b, gs_b)

        def _pair(i, carry):
            a = 2 * i
            b = 2 * i + 1
            pltpu.make_async_copy(feat_hbm.at[src_v.at[a]], rows_a,
                                  gs_a).wait()
            pltpu.sync_copy(rows_a, acc.at[dst_v.at[a]], add=True)
            pltpu.sync_copy(ones_v, dacc.at[dst_v.at[a]], add=True)
            pltpu.async_copy(feat_hbm.at[src_v.at[a + 2]], rows_a, gs_a)
            pltpu.make_async_copy(feat_hbm.at[src_v.at[b]], rows_b,
                                  gs_b).wait()
            pltpu.sync_copy(rows_b, acc.at[dst_v.at[b]], add=True)
            pltpu.sync_copy(ones_v, dacc.at[dst_v.at[b]], add=True)
            pltpu.async_copy(feat_hbm.at[src_v.at[b + 2]], rows_b, gs_b)
            return carry
        lax.fori_loop(0, HALF // 2, _pair, 0)
        pltpu.make_async_copy(feat_hbm.at[src_v.at[HALF]], rows_a,
                              gs_a).wait()
        pltpu.make_async_copy(feat_hbm.at[src_v.at[HALF + 1]], rows_b,
                              gs_b).wait()
    plsc.subcore_barrier()

    for i in range(RPT // CH):
        pltpu.sync_copy(
            acc.at[pl.ds(sid * RPT + i * CH, CH)],
            partial_hbm.at[pl.ds(cid * N_PAD + sid * RPT + i * CH, CH)])
    pltpu.sync_copy(dacc.at[pl.ds(sid * RPT, RPT)],
                    deg_hbm.at[pl.ds(cid * N_PAD + sid * RPT, RPT)])


_sc_call = pl.kernel(
    _sc_body,
    out_type=(
        jax.ShapeDtypeStruct((NC * N_PAD, D), jnp.float32),
        jax.ShapeDtypeStruct((NC * N_PAD,), jnp.float32),
    ),
    mesh=plsc.VectorSubcoreMesh(core_axis_name="c", subcore_axis_name="s"),
    scratch_types=[
        pltpu.VMEM((HALF + 2, CH), jnp.int32),
        pltpu.VMEM((HALF, CH), jnp.int32),
        pltpu.VMEM((CH, D), jnp.float32),
        pltpu.VMEM((CH, D), jnp.float32),
        pltpu.VMEM((CH,), jnp.float32),
        pltpu.VMEM((RPT,), jnp.float32),
        pltpu.VMEM_SHARED((N_PAD, D), jnp.float32),
        pltpu.VMEM_SHARED((N_PAD,), jnp.float32),
        pltpu.SemaphoreType.DMA,
        pltpu.SemaphoreType.DMA,
        pltpu.SemaphoreType.DMA,
        pltpu.SemaphoreType.DMA,
    ],
)


_BLK = 1000


def _tc_body(p0_ref, p1_ref, d0_ref, d1_ref, x_ref, wn_ref, ws_ref, b_ref,
             o_ref):
    deg = jnp.maximum(d0_ref[...] + d1_ref[...], 1.0)
    h = (p0_ref[...] + p1_ref[...]) / deg
    cdims = (((1,), (1,)), ((), ()))
    h_neigh = lax.dot_general(h, wn_ref[...], cdims,
                              preferred_element_type=jnp.float32)
    h_self = lax.dot_general(x_ref[...], ws_ref[...], cdims,
                             preferred_element_type=jnp.float32)
    o_ref[...] = h_self + h_neigh + b_ref[...]


def kernel(feat, edge_index, W_neigh, W_self, b_self):
    src = edge_index[0].astype(jnp.int32)
    dst = edge_index[1].astype(jnp.int32)
    pad = E_PAD - N_EDGES
    src_p = jnp.concatenate([src, jnp.zeros((pad,), jnp.int32)])
    dst_p = jnp.concatenate([dst, jnp.full((pad,), N_NODES, jnp.int32)])
    src_p = src_p.reshape(NW * NCHUNK, CH)
    dst_p = dst_p.reshape(NW * NCHUNK, CH)

    partial, deg = _sc_call(src_p, dst_p, feat)
    p0 = partial[:N_NODES]
    p1 = partial[N_PAD:N_PAD + N_NODES]
    d0 = deg[:N_NODES].reshape(N_NODES, 1)
    d1 = deg[N_PAD:N_PAD + N_NODES].reshape(N_NODES, 1)

    grid = (N_NODES // _BLK,)
    row_spec = pl.BlockSpec((_BLK, D), lambda i: (i, 0))
    deg_spec = pl.BlockSpec((_BLK, 1), lambda i: (i, 0))
    w_spec = pl.BlockSpec((D, D), lambda i: (0, 0))
    b_spec = pl.BlockSpec((1, D), lambda i: (0, 0))
    out = pl.pallas_call(
        _tc_body,
        grid=grid,
        in_specs=[row_spec, row_spec, deg_spec, deg_spec, row_spec,
                  w_spec, w_spec, b_spec],
        out_specs=row_spec,
        out_shape=jax.ShapeDtypeStruct((N_NODES, D), jnp.float32),
    )(p0, p1, d0, d1, feat, W_neigh, W_self, b_self.reshape(1, D))
    return out
